# baseline (speedup 1.0000x reference)
import functools

import jax
import jax.numpy as jnp
from jax import lax
from jax.experimental import pallas as pl
from jax.experimental.pallas import tpu as pltpu
from jax.experimental.pallas import tpu_sc as plsc

N_NODES = 10000
D_FEAT = 128
N_EDGES = 160000
N_REL = 3

NC, NS = 2, 16
CHUNK = 128
ROWS = 1280
RPT0 = 80
SUB = 40
PAD_E = ROWS * CHUNK - N_EDGES
ACC_ROWS = 10240
SHARE = ACC_ROWS // NS

_mesh = plsc.VectorSubcoreMesh(core_axis_name="c", subcore_axis_name="s")


@functools.partial(
    pl.kernel,
    out_type=(
        jax.ShapeDtypeStruct((N_REL, ACC_ROWS, D_FEAT), jnp.float32),
        jax.ShapeDtypeStruct((N_REL * ACC_ROWS,), jnp.float32),
    ),
    mesh=_mesh,
    scratch_types=[
        pltpu.VMEM((SUB, CHUNK), jnp.int32),
        pltpu.VMEM((SUB, CHUNK), jnp.int32),
        pltpu.VMEM((CHUNK, D_FEAT), jnp.float32),
        pltpu.VMEM((CHUNK, D_FEAT), jnp.float32),
        pltpu.VMEM((CHUNK,), jnp.float32),
        pltpu.VMEM((16, D_FEAT), jnp.float32),
        pltpu.VMEM((SHARE,), jnp.float32),
        pltpu.VMEM_SHARED((ACC_ROWS, D_FEAT), jnp.float32),
        pltpu.VMEM_SHARED((ACC_ROWS,), jnp.float32),
        pltpu.SemaphoreType.DMA,
        pltpu.SemaphoreType.DMA,
        pltpu.SemaphoreType.DMA,
        pltpu.SemaphoreType.DMA,
        pltpu.SemaphoreType.DMA,
    ],
)
def _sc_aggregate(x_hbm, src_hbm, dst_hbm, ones_hbm, zacc_hbm, zdeg_hbm,
                  acc_out, deg_out,
                  src_v, dst_v, rows_v0, rows_v1, ones_v, zacc_v, zdeg_v,
                  acc_sh, deg_sh, sem_g0, sem_g1, sem_s0, sem_s1, sem_d):
    c = lax.axis_index("c")
    s = lax.axis_index("s")

    pltpu.sync_copy(ones_hbm, ones_v)
    pltpu.sync_copy(zacc_hbm, zacc_v)
    pltpu.sync_copy(zdeg_hbm, zdeg_v)

    for r in range(N_REL):
        z0 = s * SHARE

        @pl.when(c == 0)
        def _():
            @pl.loop(0, SHARE // 16)
            def _(k):
                pltpu.sync_copy(zacc_v, acc_sh.at[pl.ds(z0 + k * 16, 16)])

            pltpu.sync_copy(zdeg_v, deg_sh.at[pl.ds(z0, SHARE)])

        plsc.subcore_barrier()

        bufs = (rows_v0, rows_v1)
        sem_g = (sem_g0, sem_g1)
        sem_s = (sem_s0, sem_s1)

        def run_block(row_base, n):
            pltpu.sync_copy(src_hbm.at[r, pl.ds(row_base, n)],
                            src_v.at[pl.ds(0, n)])
            pltpu.sync_copy(dst_hbm.at[r, pl.ds(row_base, n)],
                            dst_v.at[pl.ds(0, n)])
            for b in range(2):
                pltpu.async_copy(x_hbm.at[src_v.at[b]], bufs[b], sem_g[b])

            @pl.loop(0, n // 2)
            def _(t):
                for b in range(2):
                    j = t * 2 + b
                    pltpu.make_async_copy(
                        x_hbm.at[src_v.at[j]], bufs[b], sem_g[b]).wait()
                    pltpu.async_copy(ones_v, deg_sh.at[dst_v.at[j]], sem_d,
                                     add=True)
                    pltpu.async_copy(bufs[b], acc_sh.at[dst_v.at[j]],
                                     sem_s[b], add=True).wait()

                    @pl.when(j < n - 2)
                    def _():
                        pltpu.async_copy(
                            x_hbm.at[src_v.at[j + 2]], bufs[b], sem_g[b])

            @pl.loop(0, n)
            def _(j):
                pltpu.make_async_copy(
                    ones_v, deg_sh.at[dst_v.at[0]], sem_d).wait()

        @pl.when(c == 0)
        def _():
            run_block(s * RPT0, SUB)
            run_block(s * RPT0 + SUB, SUB)

        plsc.subcore_barrier()
        o0 = s * SHARE

        @pl.when(c == 0)
        def _():
            pltpu.sync_copy(acc_sh.at[pl.ds(o0, SHARE)],
                            acc_out.at[r, pl.ds(o0, SHARE)])
            pltpu.sync_copy(deg_sh.at[pl.ds(o0, SHARE)],
                            deg_out.at[pl.ds(r * ACC_ROWS + o0, SHARE)])

        plsc.subcore_barrier()


_BN = 1280


def _tc_body(acc_ref, deg_ref, w_ref, o_ref):
    h = jnp.zeros((_BN, D_FEAT), jnp.float32)
    for r in range(N_REL):
        a = acc_ref[r]
        d = jnp.maximum(deg_ref[r], 1.0)
        h = h + jnp.dot(a / d, w_ref[r],
                        preferred_element_type=jnp.float32,
                        precision=lax.Precision.HIGHEST)
    o_ref[...] = h


def _tc_combine(acc, deg, w):
    return pl.pallas_call(
        _tc_body,
        grid=(ACC_ROWS // _BN,),
        in_specs=[
            pl.BlockSpec((N_REL, _BN, D_FEAT), lambda i: (0, i, 0)),
            pl.BlockSpec((N_REL, _BN, 1), lambda i: (0, i, 0)),
            pl.BlockSpec((N_REL, D_FEAT, D_FEAT), lambda i: (0, 0, 0)),
        ],
        out_specs=pl.BlockSpec((_BN, D_FEAT), lambda i: (i, 0)),
        out_shape=jax.ShapeDtypeStruct((ACC_ROWS, D_FEAT), jnp.float32),
    )(acc, deg, w)


def kernel(x, edge_index_rel0, edge_index_rel1, edge_index_rel2,
           W_rel0, W_rel1, W_rel2):
    src = jnp.stack([edge_index_rel0[0], edge_index_rel1[0],
                     edge_index_rel2[0]]).astype(jnp.int32)
    dst = jnp.stack([edge_index_rel0[1], edge_index_rel1[1],
                     edge_index_rel2[1]]).astype(jnp.int32)
    src = jnp.pad(src, ((0, 0), (0, PAD_E)))
    pad_dst = jnp.broadcast_to(
        N_NODES + (jnp.arange(PAD_E, dtype=jnp.int32) % (ACC_ROWS - N_NODES)),
        (N_REL, PAD_E))
    dst = jnp.concatenate([dst, pad_dst], axis=1)
    src = src.reshape(N_REL, ROWS, CHUNK)
    dst = dst.reshape(N_REL, ROWS, CHUNK)

    ones = jnp.ones((CHUNK,), jnp.float32)
    zacc = jnp.zeros((16, D_FEAT), jnp.float32)
    zdeg = jnp.zeros((SHARE,), jnp.float32)

    acc, deg = _sc_aggregate(x, src, dst, ones, zacc, zdeg)
    deg = deg.reshape(N_REL, ACC_ROWS, 1)
    w = jnp.stack([W_rel0, W_rel1, W_rel2])
    return _tc_combine(acc, deg, w)[:N_NODES]


# speedup vs baseline: 2.1104x; 2.1104x over previous
import functools

import jax
import jax.numpy as jnp
from jax import lax
from jax.experimental import pallas as pl
from jax.experimental.pallas import tpu as pltpu
from jax.experimental.pallas import tpu_sc as plsc

N_NODES = 10000
D_FEAT = 128
N_EDGES = 160000
N_REL = 3

NC, NS = 2, 16
CHUNK = 128
ROWS = 1280
RPT0 = 80
SUB = 40
PAD_E = ROWS * CHUNK - N_EDGES
ACC_ROWS = 10240
SHARE = ACC_ROWS // NS

_mesh = plsc.VectorSubcoreMesh(core_axis_name="c", subcore_axis_name="s")


@functools.partial(
    pl.kernel,
    out_type=(
        jax.ShapeDtypeStruct((N_REL, ACC_ROWS, D_FEAT), jnp.float32),
        jax.ShapeDtypeStruct((N_REL * ACC_ROWS,), jnp.float32),
    ),
    mesh=_mesh,
    scratch_types=[
        pltpu.VMEM((SUB, CHUNK), jnp.int32),
        pltpu.VMEM((SUB, CHUNK), jnp.int32),
        pltpu.VMEM((CHUNK, D_FEAT), jnp.float32),
        pltpu.VMEM((CHUNK, D_FEAT), jnp.float32),
        pltpu.VMEM((CHUNK,), jnp.float32),
        pltpu.VMEM((16, D_FEAT), jnp.float32),
        pltpu.VMEM((SHARE,), jnp.float32),
        pltpu.VMEM_SHARED((ACC_ROWS, D_FEAT), jnp.float32),
        pltpu.VMEM_SHARED((ACC_ROWS,), jnp.float32),
        pltpu.SemaphoreType.DMA,
        pltpu.SemaphoreType.DMA,
        pltpu.SemaphoreType.DMA,
        pltpu.SemaphoreType.DMA,
        pltpu.SemaphoreType.DMA,
    ],
)
def _sc_aggregate(x_hbm, src_hbm, dst_hbm, ones_hbm, zacc_hbm, zdeg_hbm,
                  acc_out, deg_out,
                  src_v, dst_v, rows_v0, rows_v1, ones_v, zacc_v, zdeg_v,
                  acc_sh, deg_sh, sem_g0, sem_g1, sem_s0, sem_s1, sem_d):
    c = lax.axis_index("c")
    s = lax.axis_index("s")

    pltpu.sync_copy(ones_hbm, ones_v)
    pltpu.sync_copy(zacc_hbm, zacc_v)
    pltpu.sync_copy(zdeg_hbm, zdeg_v)

    for r in range(N_REL):
        z0 = s * SHARE

        @pl.when(c == 0)
        def _():
            @pl.loop(0, SHARE // 16)
            def _(k):
                pltpu.sync_copy(zacc_v, acc_sh.at[pl.ds(z0 + k * 16, 16)])

            pltpu.sync_copy(zdeg_v, deg_sh.at[pl.ds(z0, SHARE)])

        plsc.subcore_barrier()

        bufs = (rows_v0, rows_v1)
        sem_g = (sem_g0, sem_g1)
        sem_s = (sem_s0, sem_s1)

        def run_block(row_base, n):
            pltpu.sync_copy(src_hbm.at[r, pl.ds(row_base, n)],
                            src_v.at[pl.ds(0, n)])
            pltpu.sync_copy(dst_hbm.at[r, pl.ds(row_base, n)],
                            dst_v.at[pl.ds(0, n)])
            for b in range(2):
                pltpu.async_copy(x_hbm.at[src_v.at[b]], bufs[b], sem_g[b])

            @pl.loop(0, n // 2)
            def _(t):
                for b in range(2):
                    j = t * 2 + b
                    pltpu.make_async_copy(
                        x_hbm.at[src_v.at[j]], bufs[b], sem_g[b]).wait()
                    pltpu.async_copy(ones_v, deg_sh.at[dst_v.at[j]], sem_d,
                                     add=True)
                    pltpu.async_copy(bufs[b], acc_sh.at[dst_v.at[j]],
                                     sem_s[b], add=True).wait()

                    @pl.when(j < n - 2)
                    def _():
                        pltpu.async_copy(
                            x_hbm.at[src_v.at[j + 2]], bufs[b], sem_g[b])

            @pl.loop(0, n)
            def _(j):
                pltpu.make_async_copy(
                    ones_v, deg_sh.at[dst_v.at[0]], sem_d).wait()

        @pl.when(c == 0)
        def _():
            run_block(s * RPT0, SUB)
            run_block(s * RPT0 + SUB, SUB)

        plsc.subcore_barrier()
        o0 = s * SHARE

        @pl.when(c == 0)
        def _():
            pltpu.sync_copy(acc_sh.at[pl.ds(o0, SHARE)],
                            acc_out.at[r, pl.ds(o0, SHARE)])
            pltpu.sync_copy(deg_sh.at[pl.ds(o0, SHARE)],
                            deg_out.at[pl.ds(r * ACC_ROWS + o0, SHARE)])

        plsc.subcore_barrier()


_BN = 1280


def _tc_body(acc_ref, deg_ref, w_ref, o_ref):
    h = jnp.zeros((_BN, D_FEAT), jnp.float32)
    for r in range(N_REL):
        a = acc_ref[r]
        d = jnp.maximum(deg_ref[r], 1.0)
        h = h + jnp.dot(a / d, w_ref[r],
                        preferred_element_type=jnp.float32,
                        precision=lax.Precision.HIGHEST)
    o_ref[...] = h


def _tc_combine(acc, deg, w):
    return pl.pallas_call(
        _tc_body,
        grid=(ACC_ROWS // _BN,),
        in_specs=[
            pl.BlockSpec((N_REL, _BN, D_FEAT), lambda i: (0, i, 0)),
            pl.BlockSpec((N_REL, _BN, 1), lambda i: (0, i, 0)),
            pl.BlockSpec((N_REL, D_FEAT, D_FEAT), lambda i: (0, 0, 0)),
        ],
        out_specs=pl.BlockSpec((_BN, D_FEAT), lambda i: (i, 0)),
        out_shape=jax.ShapeDtypeStruct((ACC_ROWS, D_FEAT), jnp.float32),
    )(acc, deg, w)


def kernel(x, edge_index_rel0, edge_index_rel1, edge_index_rel2,
           W_rel0, W_rel1, W_rel2):
    src = jnp.stack([edge_index_rel0[0], edge_index_rel1[0],
                     edge_index_rel2[0]]).astype(jnp.int32)
    dst = jnp.stack([edge_index_rel0[1], edge_index_rel1[1],
                     edge_index_rel2[1]]).astype(jnp.int32)
    pad_src = jnp.broadcast_to(
        jnp.arange(PAD_E, dtype=jnp.int32) % N_NODES, (N_REL, PAD_E))
    src = jnp.concatenate([src, pad_src], axis=1)
    pad_dst = jnp.broadcast_to(
        N_NODES + (jnp.arange(PAD_E, dtype=jnp.int32) % (ACC_ROWS - N_NODES)),
        (N_REL, PAD_E))
    dst = jnp.concatenate([dst, pad_dst], axis=1)
    src = (src.reshape(N_REL, RPT0, NS, CHUNK).transpose(0, 2, 1, 3)
           .reshape(N_REL, ROWS, CHUNK))
    dst = (dst.reshape(N_REL, RPT0, NS, CHUNK).transpose(0, 2, 1, 3)
           .reshape(N_REL, ROWS, CHUNK))

    ones = jnp.ones((CHUNK,), jnp.float32)
    zacc = jnp.zeros((16, D_FEAT), jnp.float32)
    zdeg = jnp.zeros((SHARE,), jnp.float32)

    acc, deg = _sc_aggregate(x, src, dst, ones, zacc, zdeg)
    deg = deg.reshape(N_REL, ACC_ROWS, 1)
    w = jnp.stack([W_rel0, W_rel1, W_rel2])
    return _tc_combine(acc, deg, w)[:N_NODES]


# speedup vs baseline: 2.9891x; 1.4164x over previous
import functools

import jax
import jax.numpy as jnp
from jax import lax
from jax.experimental import pallas as pl
from jax.experimental.pallas import tpu as pltpu
from jax.experimental.pallas import tpu_sc as plsc

N_NODES = 10000
D_FEAT = 128
N_EDGES = 160000
N_REL = 3

NC, NS = 2, 16
CHUNK = 128
ROWS = 1280
RPT = 40
PAD_E = ROWS * CHUNK - N_EDGES
ACC_ROWS = 10240
SHARE = ACC_ROWS // NS

_mesh = plsc.VectorSubcoreMesh(core_axis_name="c", subcore_axis_name="s")


@functools.partial(
    pl.kernel,
    out_type=(
        jax.ShapeDtypeStruct((NC, N_REL, ACC_ROWS, D_FEAT), jnp.float32),
        jax.ShapeDtypeStruct((NC * N_REL * ACC_ROWS,), jnp.float32),
    ),
    mesh=_mesh,
    scratch_types=[
        pltpu.VMEM((RPT, CHUNK), jnp.int32),
        pltpu.VMEM((RPT, CHUNK), jnp.int32),
        pltpu.VMEM((CHUNK, D_FEAT), jnp.float32),
        pltpu.VMEM((CHUNK, D_FEAT), jnp.float32),
        pltpu.VMEM((CHUNK,), jnp.float32),
        pltpu.VMEM((16, D_FEAT), jnp.float32),
        pltpu.VMEM((SHARE,), jnp.float32),
        pltpu.VMEM_SHARED((ACC_ROWS, D_FEAT), jnp.float32),
        pltpu.VMEM_SHARED((ACC_ROWS,), jnp.float32),
        pltpu.SemaphoreType.DMA,
        pltpu.SemaphoreType.DMA,
        pltpu.SemaphoreType.DMA,
        pltpu.SemaphoreType.DMA,
        pltpu.SemaphoreType.DMA,
    ],
)
def _sc_aggregate(x_hbm, src_hbm, dst_hbm, ones_hbm, zacc_hbm, zdeg_hbm,
                  acc_out, deg_out,
                  src_v, dst_v, rows_v0, rows_v1, ones_v, zacc_v, zdeg_v,
                  acc_sh, deg_sh, sem_g0, sem_g1, sem_s0, sem_s1, sem_d):
    c = lax.axis_index("c")
    s = lax.axis_index("s")

    pltpu.sync_copy(ones_hbm, ones_v)
    pltpu.sync_copy(zacc_hbm, zacc_v)
    pltpu.sync_copy(zdeg_hbm, zdeg_v)

    wid = s * NC + c
    for r in range(N_REL):
        z0 = s * SHARE

        @pl.loop(0, SHARE // 16)
        def _(k):
            pltpu.sync_copy(zacc_v, acc_sh.at[pl.ds(z0 + k * 16, 16)])

        pltpu.sync_copy(zdeg_v, deg_sh.at[pl.ds(z0, SHARE)])
        plsc.subcore_barrier()

        bufs = (rows_v0, rows_v1)
        sem_g = (sem_g0, sem_g1)
        sem_s = (sem_s0, sem_s1)

        def run_block(row_base, n):
            pltpu.sync_copy(src_hbm.at[r, pl.ds(row_base, n)],
                            src_v.at[pl.ds(0, n)])
            pltpu.sync_copy(dst_hbm.at[r, pl.ds(row_base, n)],
                            dst_v.at[pl.ds(0, n)])
            for b in range(2):
                pltpu.async_copy(x_hbm.at[src_v.at[b]], bufs[b], sem_g[b])

            @pl.loop(0, n // 2)
            def _(t):
                for b in range(2):
                    j = t * 2 + b
                    pltpu.make_async_copy(
                        x_hbm.at[src_v.at[j]], bufs[b], sem_g[b]).wait()
                    pltpu.async_copy(ones_v, deg_sh.at[dst_v.at[j]], sem_d,
                                     add=True)
                    pltpu.async_copy(bufs[b], acc_sh.at[dst_v.at[j]],
                                     sem_s[b], add=True).wait()

                    @pl.when(j < n - 2)
                    def _():
                        pltpu.async_copy(
                            x_hbm.at[src_v.at[j + 2]], bufs[b], sem_g[b])

            @pl.loop(0, n)
            def _(j):
                pltpu.make_async_copy(
                    ones_v, deg_sh.at[dst_v.at[0]], sem_d).wait()

        run_block(wid * RPT, RPT)

        plsc.subcore_barrier()
        o0 = s * SHARE
        pltpu.sync_copy(acc_sh.at[pl.ds(o0, SHARE)],
                        acc_out.at[c, r, pl.ds(o0, SHARE)])
        pltpu.sync_copy(deg_sh.at[pl.ds(o0, SHARE)],
                        deg_out.at[pl.ds((c * N_REL + r) * ACC_ROWS + o0,
                                         SHARE)])
        plsc.subcore_barrier()


_BN = 1280


def _tc_body(acc_ref, deg_ref, w_ref, o_ref):
    h = jnp.zeros((_BN, D_FEAT), jnp.float32)
    for r in range(N_REL):
        a = acc_ref[0, r] + acc_ref[1, r]
        d = jnp.maximum(deg_ref[0, r] + deg_ref[1, r], 1.0)
        h = h + jnp.dot(a / d, w_ref[r],
                        preferred_element_type=jnp.float32,
                        precision=lax.Precision.HIGHEST)
    o_ref[...] = h


def _tc_combine(acc, deg, w):
    return pl.pallas_call(
        _tc_body,
        grid=(ACC_ROWS // _BN,),
        in_specs=[
            pl.BlockSpec((NC, N_REL, _BN, D_FEAT), lambda i: (0, 0, i, 0)),
            pl.BlockSpec((NC, N_REL, _BN, 1), lambda i: (0, 0, i, 0)),
            pl.BlockSpec((N_REL, D_FEAT, D_FEAT), lambda i: (0, 0, 0)),
        ],
        out_specs=pl.BlockSpec((_BN, D_FEAT), lambda i: (i, 0)),
        out_shape=jax.ShapeDtypeStruct((ACC_ROWS, D_FEAT), jnp.float32),
    )(acc, deg, w)


def kernel(x, edge_index_rel0, edge_index_rel1, edge_index_rel2,
           W_rel0, W_rel1, W_rel2):
    src = jnp.stack([edge_index_rel0[0], edge_index_rel1[0],
                     edge_index_rel2[0]]).astype(jnp.int32)
    dst = jnp.stack([edge_index_rel0[1], edge_index_rel1[1],
                     edge_index_rel2[1]]).astype(jnp.int32)
    pad_src = jnp.broadcast_to(
        jnp.arange(PAD_E, dtype=jnp.int32) % N_NODES, (N_REL, PAD_E))
    src = jnp.concatenate([src, pad_src], axis=1)
    pad_dst = jnp.broadcast_to(
        N_NODES + (jnp.arange(PAD_E, dtype=jnp.int32) % (ACC_ROWS - N_NODES)),
        (N_REL, PAD_E))
    dst = jnp.concatenate([dst, pad_dst], axis=1)
    src = (src.reshape(N_REL, RPT, NC * NS, CHUNK).transpose(0, 2, 1, 3)
           .reshape(N_REL, ROWS, CHUNK))
    dst = (dst.reshape(N_REL, RPT, NC * NS, CHUNK).transpose(0, 2, 1, 3)
           .reshape(N_REL, ROWS, CHUNK))

    ones = jnp.ones((CHUNK,), jnp.float32)
    zacc = jnp.zeros((16, D_FEAT), jnp.float32)
    zdeg = jnp.zeros((SHARE,), jnp.float32)

    acc, deg = _sc_aggregate(x, src, dst, ones, zacc, zdeg)
    deg = deg.reshape(NC, N_REL, ACC_ROWS, 1)
    w = jnp.stack([W_rel0, W_rel1, W_rel2])
    return _tc_combine(acc, deg, w)[:N_NODES]


# speedup vs baseline: 3.4405x; 1.1510x over previous
import functools

import jax
import jax.numpy as jnp
from jax import lax
from jax.experimental import pallas as pl
from jax.experimental.pallas import tpu as pltpu
from jax.experimental.pallas import tpu_sc as plsc

N_NODES = 10000
D_FEAT = 128
N_EDGES = 160000
N_REL = 3

NC, NS = 2, 16
CHUNK = 128
ROWS = 1280
RPT = 40
PAD_E = ROWS * CHUNK - N_EDGES
ACC_ROWS = 10240
SHARE = ACC_ROWS // NS

_mesh = plsc.VectorSubcoreMesh(core_axis_name="c", subcore_axis_name="s")


@functools.partial(
    pl.kernel,
    out_type=(
        jax.ShapeDtypeStruct((NC, N_REL, ACC_ROWS, D_FEAT), jnp.float32),
        jax.ShapeDtypeStruct((NC * N_REL * ACC_ROWS,), jnp.float32),
    ),
    mesh=_mesh,
    scratch_types=[
        pltpu.VMEM((RPT, CHUNK), jnp.int32),
        pltpu.VMEM((RPT, CHUNK), jnp.int32),
        pltpu.VMEM((CHUNK, D_FEAT), jnp.float32),
        pltpu.VMEM((CHUNK, D_FEAT), jnp.float32),
        pltpu.VMEM((CHUNK,), jnp.float32),
        pltpu.VMEM((16, D_FEAT), jnp.float32),
        pltpu.VMEM((SHARE,), jnp.float32),
        pltpu.VMEM_SHARED((ACC_ROWS, D_FEAT), jnp.float32),
        pltpu.VMEM_SHARED((ACC_ROWS,), jnp.float32),
        pltpu.SemaphoreType.DMA,
        pltpu.SemaphoreType.DMA,
        pltpu.SemaphoreType.DMA,
        pltpu.SemaphoreType.DMA,
        pltpu.SemaphoreType.DMA,
    ],
)
def _sc_aggregate(x_hbm, src_hbm, dst_hbm, ones_hbm, zacc_hbm, zdeg_hbm,
                  acc_out, deg_out,
                  src_v, dst_v, rows_v0, rows_v1, ones_v, zacc_v, zdeg_v,
                  acc_sh, deg_sh, sem_g0, sem_g1, sem_s0, sem_s1, sem_d):
    c = lax.axis_index("c")
    s = lax.axis_index("s")

    pltpu.sync_copy(ones_hbm, ones_v)
    pltpu.sync_copy(zacc_hbm, zacc_v)
    pltpu.sync_copy(zdeg_hbm, zdeg_v)

    wid = s * NC + c
    for r in range(N_REL):
        z0 = s * SHARE

        @pl.loop(0, SHARE // 16)
        def _(k):
            pltpu.sync_copy(zacc_v, acc_sh.at[pl.ds(z0 + k * 16, 16)])

        pltpu.sync_copy(zdeg_v, deg_sh.at[pl.ds(z0, SHARE)])
        plsc.subcore_barrier()

        bufs = (rows_v0, rows_v1)
        sem_g = (sem_g0, sem_g1)
        sem_s = (sem_s0, sem_s1)

        def run_block(row_base, n):
            pltpu.sync_copy(src_hbm.at[r, pl.ds(row_base, n)],
                            src_v.at[pl.ds(0, n)])
            pltpu.sync_copy(dst_hbm.at[r, pl.ds(row_base, n)],
                            dst_v.at[pl.ds(0, n)])
            for b in range(2):
                pltpu.async_copy(x_hbm.at[src_v.at[b]], bufs[b], sem_g[b])

            @pl.loop(0, n // 2)
            def _(t):
                for b in range(2):
                    j = t * 2 + b
                    pltpu.make_async_copy(
                        x_hbm.at[src_v.at[j]], bufs[b], sem_g[b]).wait()
                    pltpu.async_copy(ones_v, deg_sh.at[dst_v.at[j]], sem_d,
                                     add=True)
                    pltpu.async_copy(bufs[b], acc_sh.at[dst_v.at[j]],
                                     sem_s[b], add=True).wait()

                    @pl.when(j < n - 2)
                    def _():
                        pltpu.async_copy(
                            x_hbm.at[src_v.at[j + 2]], bufs[b], sem_g[b])

            @pl.loop(0, n)
            def _(j):
                pltpu.make_async_copy(
                    ones_v, deg_sh.at[dst_v.at[0]], sem_d).wait()

        run_block(wid * RPT, RPT)

        plsc.subcore_barrier()
        o0 = s * SHARE
        pltpu.sync_copy(acc_sh.at[pl.ds(o0, SHARE)],
                        acc_out.at[c, r, pl.ds(o0, SHARE)])
        pltpu.sync_copy(deg_sh.at[pl.ds(o0, SHARE)],
                        deg_out.at[pl.ds((c * N_REL + r) * ACC_ROWS + o0,
                                         SHARE)])
        plsc.subcore_barrier()


_BN = 2048


def _tc_body(acc_ref, deg_ref, w_ref, o_ref):
    h = jnp.zeros((_BN, D_FEAT), jnp.float32)
    n_stripes = _BN // 128
    for r in range(N_REL):
        a = acc_ref[0, r] + acc_ref[1, r]
        d = jnp.maximum(deg_ref[0, r] + deg_ref[1, r], 1.0)
        dt = jnp.transpose(d)
        scaled = jnp.concatenate(
            [a[k * 128:(k + 1) * 128, :] / dt[:, k:k + 1]
             for k in range(n_stripes)], axis=0)
        h = h + jnp.dot(scaled, w_ref[r],
                        preferred_element_type=jnp.float32,
                        precision=lax.Precision.HIGHEST)
    o_ref[...] = h


def _tc_combine(acc, deg, w):
    return pl.pallas_call(
        _tc_body,
        grid=(ACC_ROWS // _BN,),
        in_specs=[
            pl.BlockSpec((NC, N_REL, _BN, D_FEAT), lambda i: (0, 0, i, 0)),
            pl.BlockSpec((NC, N_REL, _BN // 128, 128), lambda i: (0, 0, i, 0)),
            pl.BlockSpec((N_REL, D_FEAT, D_FEAT), lambda i: (0, 0, 0)),
        ],
        out_specs=pl.BlockSpec((_BN, D_FEAT), lambda i: (i, 0)),
        out_shape=jax.ShapeDtypeStruct((ACC_ROWS, D_FEAT), jnp.float32),
    )(acc, deg, w)


def kernel(x, edge_index_rel0, edge_index_rel1, edge_index_rel2,
           W_rel0, W_rel1, W_rel2):
    src = jnp.stack([edge_index_rel0[0], edge_index_rel1[0],
                     edge_index_rel2[0]]).astype(jnp.int32)
    dst = jnp.stack([edge_index_rel0[1], edge_index_rel1[1],
                     edge_index_rel2[1]]).astype(jnp.int32)
    pad_src = jnp.broadcast_to(
        jnp.arange(PAD_E, dtype=jnp.int32) % N_NODES, (N_REL, PAD_E))
    src = jnp.concatenate([src, pad_src], axis=1)
    pad_dst = jnp.broadcast_to(
        N_NODES + (jnp.arange(PAD_E, dtype=jnp.int32) % (ACC_ROWS - N_NODES)),
        (N_REL, PAD_E))
    dst = jnp.concatenate([dst, pad_dst], axis=1)
    src = (src.reshape(N_REL, RPT, NC * NS, CHUNK).transpose(0, 2, 1, 3)
           .reshape(N_REL, ROWS, CHUNK))
    dst = (dst.reshape(N_REL, RPT, NC * NS, CHUNK).transpose(0, 2, 1, 3)
           .reshape(N_REL, ROWS, CHUNK))

    ones = jnp.ones((CHUNK,), jnp.float32)
    zacc = jnp.zeros((16, D_FEAT), jnp.float32)
    zdeg = jnp.zeros((SHARE,), jnp.float32)

    acc, deg = _sc_aggregate(x, src, dst, ones, zacc, zdeg)
    deg = deg.reshape(NC, N_REL, ACC_ROWS // 128, 128)
    w = jnp.stack([W_rel0, W_rel1, W_rel2])
    return _tc_combine(acc, deg, w)[:N_NODES]
